# Initial kernel scaffold; baseline (speedup 1.0000x reference)
import jax
import jax.numpy as jnp
from jax.experimental import pallas as pl


def kernel(locIndexes, l_input, P_embed, L_embed, W1, b1, W2, b2, W3, b3):
    raise NotImplementedError("write your pallas kernel here")


# speedup vs baseline: 18.1913x; 18.1913x over previous
import jax
import jax.numpy as jnp
from jax import lax
from jax.experimental import pallas as pl
from jax.experimental.pallas import tpu as pltpu
from jax.experimental.pallas import tpu_sc as plsc

N_ROWS = 100001
H = 128
BLK = 2048
NBLK = 49
NPAD = NBLK * BLK
NW = 32
E = 4096 * 200
E_PER_W = E // NW
CHUNK = 12800
NCHUNK = E_PER_W // CHUNK
MASK_PER_W = NPAD // NW
NLOC = 4096


def _sc_hist_body(l_hbm, loc_hbm, partials_hbm, maskcnt_hbm,
                  counts_v, idx_v, mask_v, loc_v):
    wid = lax.axis_index("s") * 2 + lax.axis_index("c")
    zeros16 = jnp.zeros((16,), jnp.float32)

    def zero_counts(i, c):
        counts_v[pl.ds(i * 16, 16)] = zeros16
        return c
    lax.fori_loop(0, NPAD // 16, zero_counts, 0, unroll=8)

    base_e = wid * E_PER_W
    for c in range(NCHUNK):
        pltpu.sync_copy(l_hbm.at[pl.ds(base_e + c * CHUNK, CHUNK)], idx_v)

        def hist(j, c_):
            idx = idx_v[pl.ds(j * 16, 16)]
            cnt, last = plsc.scan_count(idx)
            plsc.addupdate_scatter(counts_v, [idx], cnt.astype(jnp.float32),
                                   mask=last)
            return c_
        lax.fori_loop(0, CHUNK // 16, hist, 0, unroll=4)
    pltpu.sync_copy(counts_v, partials_hbm.at[pl.ds(wid * NPAD, NPAD)])

    def zero_mask(i, c):
        mask_v[pl.ds(i * 16, 16)] = zeros16
        return c
    lax.fori_loop(0, MASK_PER_W // 16, zero_mask, 0, unroll=8)
    pltpu.sync_copy(loc_hbm, loc_v)
    mbase = wid * MASK_PER_W

    def mloop(j, c_):
        idx = loc_v[pl.ds(j * 16, 16)]
        cnt, last = plsc.scan_count(idx)
        inr = (idx >= mbase) & (idx < mbase + MASK_PER_W)
        plsc.addupdate_scatter(mask_v, [idx - mbase], cnt.astype(jnp.float32),
                               mask=last & inr)
        return c_
    lax.fori_loop(0, NLOC // 16, mloop, 0, unroll=4)
    pltpu.sync_copy(mask_v, maskcnt_hbm.at[pl.ds(mbase, MASK_PER_W)])


def _sc_hist(l_flat, loc_idx):
    mesh = plsc.VectorSubcoreMesh(core_axis_name="c", subcore_axis_name="s")
    return pl.kernel(
        _sc_hist_body,
        out_type=(jax.ShapeDtypeStruct((NW * NPAD,), jnp.float32),
                  jax.ShapeDtypeStruct((NPAD,), jnp.float32)),
        mesh=mesh,
        scratch_types=[
            pltpu.VMEM((NPAD,), jnp.float32),
            pltpu.VMEM((CHUNK,), jnp.int32),
            pltpu.VMEM((MASK_PER_W,), jnp.float32),
            pltpu.VMEM((NLOC,), jnp.int32),
        ],
        compiler_params=pltpu.CompilerParams(needs_layout_passes=False),
    )(l_flat, loc_idx)


_DN_ROWMAT = (((1,), (0,)), ((), ()))
_DN_OUTER = (((0,), (0,)), ((), ()))


def _k1_body(part_ref, p_ref, w1_ref, b1_ref, w2_ref, b2_ref, w3_ref, b3_ref,
             out_ref, emb_ref, acc_ref):
    j = pl.program_id(0)

    @pl.when(j == 0)
    def _():
        acc_ref[...] = jnp.zeros_like(acc_ref)

    w = jnp.sum(part_ref[...], axis=0, keepdims=True)
    rows = j * BLK + lax.broadcasted_iota(jnp.int32, (BLK, H), 0)
    pb = jnp.where(rows < N_ROWS, p_ref[...], 0.0)
    acc_ref[...] += lax.dot_general(w, pb, _DN_ROWMAT,
                                    preferred_element_type=jnp.float32)

    @pl.when(j == NBLK - 1)
    def _():
        e = acc_ref[...]
        emb_ref[...] = e
        x = lax.dot_general(e, w1_ref[...], _DN_ROWMAT,
                            preferred_element_type=jnp.float32) + b1_ref[...]
        x = jnp.maximum(x, 0.0)
        x = lax.dot_general(x, w2_ref[...], _DN_ROWMAT,
                            preferred_element_type=jnp.float32) + b2_ref[...]
        x = jnp.maximum(x, 0.0)
        z = lax.dot_general(x, w3_ref[...], _DN_ROWMAT,
                            preferred_element_type=jnp.float32) + b3_ref[...]
        out_ref[...] = 1.0 / (1.0 + jnp.exp(-z))


def _k1(partials, p_embed, w1, b1, w2, b2, w3, b3):
    full = lambda s: pl.BlockSpec(s, lambda j: (0, 0))
    return pl.pallas_call(
        _k1_body,
        grid=(NBLK,),
        in_specs=[
            pl.BlockSpec((NW, BLK), lambda j: (0, j)),
            pl.BlockSpec((BLK, H), lambda j: (j, 0)),
            full((H, H // 2)), full((1, H // 2)),
            full((H // 2, H // 4)), full((1, H // 4)),
            full((H // 4, 1)), full((1, 1)),
        ],
        out_specs=[full((1, 1)), full((1, H))],
        out_shape=[jax.ShapeDtypeStruct((1, 1), jnp.float32),
                   jax.ShapeDtypeStruct((1, H), jnp.float32)],
        scratch_shapes=[pltpu.VMEM((1, H), jnp.float32)],
        compiler_params=pltpu.CompilerParams(
            dimension_semantics=("arbitrary",)),
    )(partials, p_embed, w1, b1, w2, b2, w3, b3)


def _k2_body(m_ref, e_ref, l_ref, out_ref):
    m = (m_ref[0] > 0.0).astype(jnp.float32)
    e2d = lax.dot_general(m, e_ref[...], _DN_OUTER,
                          preferred_element_type=jnp.float32)
    m2d = lax.dot_general(m, jnp.ones((1, H), jnp.float32), _DN_OUTER,
                          preferred_element_type=jnp.float32)
    out_ref[...] = l_ref[...] * (1.0 - m2d) + e2d


def _k2(mask3d, emb, l_embed):
    return pl.pallas_call(
        _k2_body,
        grid=(NBLK,),
        in_specs=[
            pl.BlockSpec((1, 1, BLK), lambda j: (j, 0, 0)),
            pl.BlockSpec((1, H), lambda j: (0, 0)),
            pl.BlockSpec((BLK, H), lambda j: (j, 0)),
        ],
        out_specs=pl.BlockSpec((BLK, H), lambda j: (j, 0)),
        out_shape=jax.ShapeDtypeStruct((N_ROWS, H), jnp.float32),
        compiler_params=pltpu.CompilerParams(
            dimension_semantics=("parallel",)),
    )(mask3d, emb, l_embed)


def kernel(locIndexes, l_input, P_embed, L_embed, W1, b1, W2, b2, W3, b3):
    loc = locIndexes.astype(jnp.int32)
    l_flat = l_input.reshape(-1).astype(jnp.int32)
    partials_flat, maskcnt = _sc_hist(l_flat, loc)
    out11, emb = _k1(partials_flat.reshape(NW, NPAD), P_embed,
                     W1, b1.reshape(1, -1), W2, b2.reshape(1, -1),
                     W3, b3.reshape(1, 1))
    l_new = _k2(maskcnt.reshape(NBLK, 1, BLK), emb, L_embed)
    return (out11.reshape(()), l_new)
